# baseline (speedup 1.0000x reference)
import jax
import jax.numpy as jnp
from jax.experimental import pallas as pl
from jax.experimental.pallas import tpu as pltpu

_CR = 512
_NBUF = 24
_LAG = 2


def _stream_add_body(e_hbm, l_hbm, o_hbm, buf, lut_v, in_sem, out_sem, lut_sem):
    n_rows = e_hbm.shape[0]
    seq = l_hbm.shape[0]
    batch = n_rows // seq
    lut_chunks = seq // _CR
    n_chunks = n_rows // _CR

    def row0(g):
        st, b = divmod(g, batch)
        return b * seq + st * _CR

    def start_lut(st):
        pltpu.make_async_copy(
            l_hbm.at[pl.ds(st * _CR, _CR)], lut_v.at[st % 2], lut_sem.at[st % 2]
        ).start()

    def start_in(g):
        pltpu.make_async_copy(
            e_hbm.at[pl.ds(row0(g), _CR)], buf.at[g % _NBUF], in_sem.at[g % _NBUF]
        ).start()

    def out_copy(g):
        return pltpu.make_async_copy(
            buf.at[g % _NBUF], o_hbm.at[pl.ds(row0(g), _CR)], out_sem.at[g % _NBUF]
        )

    start_lut(0)
    if lut_chunks > 1:
        start_lut(1)
    for g in range(_NBUF):
        start_in(g)

    for g in range(n_chunks):
        slot = g % _NBUF
        st, b = divmod(g, batch)
        pltpu.make_async_copy(
            e_hbm.at[pl.ds(row0(g), _CR)], buf.at[slot], in_sem.at[slot]
        ).wait()
        if b == 0:
            pltpu.make_async_copy(
                l_hbm.at[pl.ds(st * _CR, _CR)], lut_v.at[st % 2], lut_sem.at[st % 2]
            ).wait()
        buf[slot] = buf[slot] + lut_v[st % 2]
        out_copy(g).start()
        if b == batch - 1 and st + 2 < lut_chunks:
            start_lut(st + 2)
        h = g - _LAG
        if h >= 0 and h + _NBUF < n_chunks:
            out_copy(h).wait()
            start_in(h + _NBUF)

    for g in range(n_chunks):
        if not (0 <= g <= n_chunks - 1 - _LAG and g + _NBUF < n_chunks):
            out_copy(g).wait()


def kernel(embed, learn_lut):
    B, S, D = embed.shape
    out2d = pl.pallas_call(
        _stream_add_body,
        in_specs=[
            pl.BlockSpec(memory_space=pltpu.MemorySpace.HBM),
            pl.BlockSpec(memory_space=pltpu.MemorySpace.HBM),
        ],
        out_specs=pl.BlockSpec(memory_space=pltpu.MemorySpace.HBM),
        out_shape=jax.ShapeDtypeStruct((B * S, D), embed.dtype),
        scratch_shapes=[
            pltpu.VMEM((_NBUF, _CR, D), jnp.float32),
            pltpu.VMEM((2, _CR, D), jnp.float32),
            pltpu.SemaphoreType.DMA((_NBUF,)),
            pltpu.SemaphoreType.DMA((_NBUF,)),
            pltpu.SemaphoreType.DMA((2,)),
        ],
    )(embed.reshape(B * S, D), learn_lut[:S])
    return out2d.reshape(B, S, D)


# speedup vs baseline: 1.0096x; 1.0096x over previous
import jax
import jax.numpy as jnp
from jax.experimental import pallas as pl
from jax.experimental.pallas import tpu as pltpu

_CR = 2048
_NBUF = 5
_LAG = 2


def _stream_add_body(e_hbm, l_hbm, o_hbm, buf, lut_v, in_sem, out_sem, lut_sem):
    n_rows = e_hbm.shape[0]
    seq = l_hbm.shape[0]
    batch = n_rows // seq
    lut_chunks = seq // _CR
    n_chunks = n_rows // _CR

    def row0(g):
        st, b = divmod(g, batch)
        return b * seq + st * _CR

    def start_lut(st):
        pltpu.make_async_copy(
            l_hbm.at[pl.ds(st * _CR, _CR)], lut_v.at[st % 2], lut_sem.at[st % 2]
        ).start()

    def start_in(g):
        pltpu.make_async_copy(
            e_hbm.at[pl.ds(row0(g), _CR)], buf.at[g % _NBUF], in_sem.at[g % _NBUF]
        ).start()

    def out_copy(g):
        return pltpu.make_async_copy(
            buf.at[g % _NBUF], o_hbm.at[pl.ds(row0(g), _CR)], out_sem.at[g % _NBUF]
        )

    start_lut(0)
    if lut_chunks > 1:
        start_lut(1)
    for g in range(_NBUF):
        start_in(g)

    for g in range(n_chunks):
        slot = g % _NBUF
        st, b = divmod(g, batch)
        pltpu.make_async_copy(
            e_hbm.at[pl.ds(row0(g), _CR)], buf.at[slot], in_sem.at[slot]
        ).wait()
        if b == 0:
            pltpu.make_async_copy(
                l_hbm.at[pl.ds(st * _CR, _CR)], lut_v.at[st % 2], lut_sem.at[st % 2]
            ).wait()
        buf[slot] = buf[slot] + lut_v[st % 2]
        out_copy(g).start()
        if b == batch - 1 and st + 2 < lut_chunks:
            start_lut(st + 2)
        h = g - _LAG
        if h >= 0 and h + _NBUF < n_chunks:
            out_copy(h).wait()
            start_in(h + _NBUF)

    for g in range(n_chunks):
        if not (0 <= g <= n_chunks - 1 - _LAG and g + _NBUF < n_chunks):
            out_copy(g).wait()


def kernel(embed, learn_lut):
    B, S, D = embed.shape
    out2d = pl.pallas_call(
        _stream_add_body,
        in_specs=[
            pl.BlockSpec(memory_space=pltpu.MemorySpace.HBM),
            pl.BlockSpec(memory_space=pltpu.MemorySpace.HBM),
        ],
        out_specs=pl.BlockSpec(memory_space=pltpu.MemorySpace.HBM),
        out_shape=jax.ShapeDtypeStruct((B * S, D), embed.dtype),
        scratch_shapes=[
            pltpu.VMEM((_NBUF, _CR, D), jnp.float32),
            pltpu.VMEM((2, _CR, D), jnp.float32),
            pltpu.SemaphoreType.DMA((_NBUF,)),
            pltpu.SemaphoreType.DMA((_NBUF,)),
            pltpu.SemaphoreType.DMA((2,)),
        ],
    )(embed.reshape(B * S, D), learn_lut[:S])
    return out2d.reshape(B, S, D)


# speedup vs baseline: 1.0214x; 1.0117x over previous
import jax
import jax.numpy as jnp
from jax.experimental import pallas as pl
from jax.experimental.pallas import tpu as pltpu

_CR = 1024
_NBUF = 12
_LAG = 1


def _stream_add_body(e_hbm, l_hbm, o_hbm, buf, lut_v, in_sem, out_sem, lut_sem):
    n_rows = e_hbm.shape[0]
    seq = l_hbm.shape[0]
    batch = n_rows // seq
    lut_chunks = seq // _CR
    n_chunks = n_rows // _CR

    def row0(g):
        st, b = divmod(g, batch)
        return b * seq + st * _CR

    def start_lut(st):
        pltpu.make_async_copy(
            l_hbm.at[pl.ds(st * _CR, _CR)], lut_v.at[st % 2], lut_sem.at[st % 2]
        ).start()

    def start_in(g):
        pltpu.make_async_copy(
            e_hbm.at[pl.ds(row0(g), _CR)], buf.at[g % _NBUF], in_sem.at[g % _NBUF]
        ).start()

    def out_copy(g):
        return pltpu.make_async_copy(
            buf.at[g % _NBUF], o_hbm.at[pl.ds(row0(g), _CR)], out_sem.at[g % _NBUF]
        )

    start_lut(0)
    if lut_chunks > 1:
        start_lut(1)
    for g in range(_NBUF):
        start_in(g)

    for g in range(n_chunks):
        slot = g % _NBUF
        st, b = divmod(g, batch)
        pltpu.make_async_copy(
            e_hbm.at[pl.ds(row0(g), _CR)], buf.at[slot], in_sem.at[slot]
        ).wait()
        if b == 0:
            pltpu.make_async_copy(
                l_hbm.at[pl.ds(st * _CR, _CR)], lut_v.at[st % 2], lut_sem.at[st % 2]
            ).wait()
        buf[slot] = buf[slot] + lut_v[st % 2]
        out_copy(g).start()
        if b == batch - 1 and st + 2 < lut_chunks:
            start_lut(st + 2)
        h = g - _LAG
        if h >= 0 and h + _NBUF < n_chunks:
            out_copy(h).wait()
            start_in(h + _NBUF)

    for g in range(n_chunks):
        if not (0 <= g <= n_chunks - 1 - _LAG and g + _NBUF < n_chunks):
            out_copy(g).wait()


def kernel(embed, learn_lut):
    B, S, D = embed.shape
    out2d = pl.pallas_call(
        _stream_add_body,
        in_specs=[
            pl.BlockSpec(memory_space=pltpu.MemorySpace.HBM),
            pl.BlockSpec(memory_space=pltpu.MemorySpace.HBM),
        ],
        out_specs=pl.BlockSpec(memory_space=pltpu.MemorySpace.HBM),
        out_shape=jax.ShapeDtypeStruct((B * S, D), embed.dtype),
        scratch_shapes=[
            pltpu.VMEM((_NBUF, _CR, D), jnp.float32),
            pltpu.VMEM((2, _CR, D), jnp.float32),
            pltpu.SemaphoreType.DMA((_NBUF,)),
            pltpu.SemaphoreType.DMA((_NBUF,)),
            pltpu.SemaphoreType.DMA((2,)),
        ],
    )(embed.reshape(B * S, D), learn_lut[:S])
    return out2d.reshape(B, S, D)


# speedup vs baseline: 1.0227x; 1.0013x over previous
import jax
import jax.numpy as jnp
from jax.experimental import pallas as pl
from jax.experimental.pallas import tpu as pltpu

_CR = 1024
_NBUF = 13
_LAG = 1


def _stream_add_body(e_hbm, l_hbm, o_hbm, buf, lut_v, in_sem, out_sem, lut_sem):
    n_rows = e_hbm.shape[0]
    seq = l_hbm.shape[0]
    batch = n_rows // seq
    lut_chunks = seq // _CR
    n_chunks = n_rows // _CR

    def row0(g):
        st, b = divmod(g, batch)
        return b * seq + st * _CR

    def start_lut(st):
        pltpu.make_async_copy(
            l_hbm.at[pl.ds(st * _CR, _CR)], lut_v.at[st % 2], lut_sem.at[st % 2]
        ).start()

    def start_in(g):
        pltpu.make_async_copy(
            e_hbm.at[pl.ds(row0(g), _CR)], buf.at[g % _NBUF], in_sem.at[g % _NBUF]
        ).start()

    def out_copy(g):
        return pltpu.make_async_copy(
            buf.at[g % _NBUF], o_hbm.at[pl.ds(row0(g), _CR)], out_sem.at[g % _NBUF]
        )

    start_lut(0)
    if lut_chunks > 1:
        start_lut(1)
    for g in range(_NBUF):
        start_in(g)

    for g in range(n_chunks):
        slot = g % _NBUF
        st, b = divmod(g, batch)
        pltpu.make_async_copy(
            e_hbm.at[pl.ds(row0(g), _CR)], buf.at[slot], in_sem.at[slot]
        ).wait()
        if b == 0:
            pltpu.make_async_copy(
                l_hbm.at[pl.ds(st * _CR, _CR)], lut_v.at[st % 2], lut_sem.at[st % 2]
            ).wait()
        buf[slot] = buf[slot] + lut_v[st % 2]
        out_copy(g).start()
        if b == batch - 1 and st + 2 < lut_chunks:
            start_lut(st + 2)
        h = g - _LAG
        if h >= 0 and h + _NBUF < n_chunks:
            out_copy(h).wait()
            start_in(h + _NBUF)

    for g in range(n_chunks):
        if not (0 <= g <= n_chunks - 1 - _LAG and g + _NBUF < n_chunks):
            out_copy(g).wait()


def kernel(embed, learn_lut):
    B, S, D = embed.shape
    out2d = pl.pallas_call(
        _stream_add_body,
        in_specs=[
            pl.BlockSpec(memory_space=pltpu.MemorySpace.HBM),
            pl.BlockSpec(memory_space=pltpu.MemorySpace.HBM),
        ],
        out_specs=pl.BlockSpec(memory_space=pltpu.MemorySpace.HBM),
        out_shape=jax.ShapeDtypeStruct((B * S, D), embed.dtype),
        compiler_params=pltpu.CompilerParams(vmem_limit_bytes=67043328),
        scratch_shapes=[
            pltpu.VMEM((_NBUF, _CR, D), jnp.float32),
            pltpu.VMEM((2, _CR, D), jnp.float32),
            pltpu.SemaphoreType.DMA((_NBUF,)),
            pltpu.SemaphoreType.DMA((_NBUF,)),
            pltpu.SemaphoreType.DMA((2,)),
        ],
    )(embed.reshape(B * S, D), learn_lut[:S])
    return out2d.reshape(B, S, D)
